# Initial kernel scaffold; baseline (speedup 1.0000x reference)
import jax
import jax.numpy as jnp
from jax.experimental import pallas as pl


def kernel(input_ids, weight):
    raise NotImplementedError("write your pallas kernel here")


# speedup vs baseline: 6.8722x; 6.8722x over previous
import functools

import jax
import jax.numpy as jnp
from jax import lax
from jax.experimental import pallas as pl
from jax.experimental.pallas import tpu as pltpu
from jax.experimental.pallas import tpu_sc as plsc

EMBED = 32
IDXW = 128
CHUNK_IR = 8


def _gather_rows(table, idx2d):
    nr = idx2d.shape[0]
    info = plsc.get_sparse_core_info()
    nw = info.num_cores * info.num_subcores
    rpw = nr // nw
    nch = rpw // CHUNK_IR
    rows_per_chunk = CHUNK_IR * IDXW

    mesh = plsc.VectorSubcoreMesh(core_axis_name="c", subcore_axis_name="s")

    @functools.partial(
        pl.kernel,
        mesh=mesh,
        compiler_params=pltpu.CompilerParams(use_tc_tiling_on_sc=False),
        out_type=jax.ShapeDtypeStruct((nr * IDXW, EMBED), jnp.float32),
        scratch_types=[
            pltpu.VMEM((CHUNK_IR, IDXW), jnp.int32),
            pltpu.VMEM((rows_per_chunk, EMBED), jnp.float32),
            pltpu.SemaphoreType.DMA,
        ],
    )
    def k(tab_hbm, idx_hbm, out_hbm, idx_v, rows_v, sem):
        wid = lax.axis_index("s") * info.num_cores + lax.axis_index("c")
        r0 = wid * rpw

        def body(g, carry):
            base = r0 + g * CHUNK_IR
            pltpu.sync_copy(idx_hbm.at[pl.ds(base, CHUNK_IR)], idx_v)
            cps = [
                pltpu.async_copy(
                    tab_hbm.at[idx_v.at[j]],
                    rows_v.at[pl.ds(j * IDXW, IDXW)],
                    sem,
                )
                for j in range(CHUNK_IR)
            ]
            for cp in cps:
                cp.wait()
            pltpu.sync_copy(
                rows_v, out_hbm.at[pl.ds(base * IDXW, rows_per_chunk)]
            )
            return carry

        lax.fori_loop(0, nch, body, 0)

    return k(table, idx2d)


def _f16_bits(o):
    u = lax.bitcast_convert_type(o, jnp.int32)
    sign = (u >> 16) & 0x8000
    au = u & 0x7FFFFFFF
    b = au - 0x38000000
    r = b >> 13
    low = b & 0x1FFF
    rup = (low > 0x1000) | ((low == 0x1000) & ((r & 1) == 1))
    hn = r + rup.astype(jnp.int32)
    xs = jnp.abs(o) * 16777216.0
    hs = jnp.round(xs).astype(jnp.int32)
    return jnp.where(au < 0x38800000, hs, hn) | sign


def _dequant(rows):
    b = rows.shape[0]
    blk = 8192

    def body(x_ref, o_ref):
        w = x_ref[...]
        m = jnp.max(jnp.abs(w), axis=1, keepdims=True)
        s = jnp.maximum(m, 1e-8) / 127.0
        q = jnp.clip(jnp.round(w / s), -127.0, 127.0)
        t = s * 8193.0
        sf = t - (t - s)
        o = q * sf
        o_ref[...] = _f16_bits(o).astype(jnp.int16)

    return pl.pallas_call(
        body,
        grid=(b // blk,),
        in_specs=[pl.BlockSpec((blk, EMBED), lambda i: (i, 0))],
        out_specs=pl.BlockSpec((blk, EMBED), lambda i: (i, 0)),
        out_shape=jax.ShapeDtypeStruct((b, EMBED), jnp.int16),
    )(rows)


def kernel(input_ids, weight):
    bsz, hist = input_ids.shape
    idx2d = input_ids.reshape(-1, IDXW)
    rows = _gather_rows(weight, idx2d)
    bits = _dequant(rows)
    out = lax.bitcast_convert_type(bits, jnp.float16)
    return out.reshape(bsz, hist, EMBED)


# speedup vs baseline: 9.6769x; 1.4081x over previous
import functools

import jax
import jax.numpy as jnp
from jax import lax
from jax.experimental import pallas as pl
from jax.experimental.pallas import tpu as pltpu
from jax.experimental.pallas import tpu_sc as plsc

EMBED = 32
IDXW = 128
CHUNK_IR = 8


def _gather_rows(table, idx2d):
    nr = idx2d.shape[0]
    info = plsc.get_sparse_core_info()
    nw = info.num_cores * info.num_subcores
    rpw = nr // nw
    nch = rpw // CHUNK_IR

    mesh = plsc.VectorSubcoreMesh(core_axis_name="c", subcore_axis_name="s")

    @functools.partial(
        pl.kernel,
        mesh=mesh,
        compiler_params=pltpu.CompilerParams(use_tc_tiling_on_sc=False),
        out_type=jax.ShapeDtypeStruct((nr * IDXW, EMBED), jnp.float32),
        scratch_types=[
            pltpu.VMEM((CHUNK_IR, IDXW), jnp.int32),
            pltpu.VMEM((CHUNK_IR * IDXW, EMBED), jnp.float32),
            pltpu.SemaphoreType.DMA,
        ],
    )
    def k(tab_hbm, idx_hbm, out_hbm, idx_v, rows_v, sem):
        wid = lax.axis_index("s") * info.num_cores + lax.axis_index("c")
        r0 = wid * rpw

        def body(g, carry):
            base = r0 + g * CHUNK_IR
            pltpu.sync_copy(idx_hbm.at[pl.ds(base, CHUNK_IR)], idx_v)
            cps = [
                pltpu.async_copy(
                    tab_hbm.at[idx_v.at[j]],
                    rows_v.at[pl.ds(j * IDXW, IDXW)],
                    sem,
                )
                for j in range(CHUNK_IR)
            ]
            for cp in cps:
                cp.wait()
            pltpu.sync_copy(
                rows_v,
                out_hbm.at[pl.ds(base * IDXW, CHUNK_IR * IDXW)],
            )
            return carry

        lax.fori_loop(0, nch, body, 0)

    return k(table, idx2d)


def _f16_bits(o):
    u = lax.bitcast_convert_type(o, jnp.int32)
    sign = (u >> 16) & 0x8000
    au = u & 0x7FFFFFFF
    b = au - 0x38000000
    r = b >> 13
    low = b & 0x1FFF
    rup = (low > 0x1000) | ((low == 0x1000) & ((r & 1) == 1))
    hn = r + rup.astype(jnp.int32)
    xs = jnp.abs(o) * 16777216.0
    hs = jnp.round(xs).astype(jnp.int32)
    return jnp.where(au < 0x38800000, hs, hn) | sign


def _seg_max32(a):
    n = a.shape[-1]
    lanes = lax.broadcasted_iota(jnp.int32, a.shape, len(a.shape) - 1) % 32
    m = a
    for o in (1, 2, 4, 8, 16):
        r1 = pltpu.roll(m, n - o, axis=len(a.shape) - 1)
        r2 = pltpu.roll(m, 32 - o, axis=len(a.shape) - 1)
        m = jnp.maximum(m, jnp.where(lanes < 32 - o, r1, r2))
    return m


def _dequant(rows4k):
    n = rows4k.shape[0]
    blk = 64

    def body(x_ref, o_ref):
        w = x_ref[...]
        m = _seg_max32(jnp.abs(w))
        s = jnp.maximum(m, 1e-8) / 127.0
        q = jnp.clip(jnp.round(w / s), -127.0, 127.0)
        t = s * 8193.0
        sf = t - (t - s)
        o = q * sf
        o_ref[...] = _f16_bits(o).astype(jnp.int16)

    return pl.pallas_call(
        body,
        grid=(n // blk,),
        in_specs=[pl.BlockSpec((blk, 4096), lambda i: (i, 0))],
        out_specs=pl.BlockSpec((blk, 4096), lambda i: (i, 0)),
        out_shape=jax.ShapeDtypeStruct((n, 4096), jnp.int16),
    )(rows4k)


def kernel(input_ids, weight):
    bsz, hist = input_ids.shape
    idx2d = input_ids.reshape(-1, IDXW)
    rows = _gather_rows(weight, idx2d)
    bits = _dequant(rows.reshape(-1, IDXW * EMBED))
    out = lax.bitcast_convert_type(bits, jnp.float16)
    return out.reshape(bsz, hist, EMBED)


# speedup vs baseline: 12.8091x; 1.3237x over previous
import functools

import jax
import jax.numpy as jnp
from jax import lax
from jax.experimental import pallas as pl
from jax.experimental.pallas import tpu as pltpu
from jax.experimental.pallas import tpu_sc as plsc

EMBED = 32
IDXW = 128
CHUNK_IR = 8


def _gather_rows(table, idx2d):
    nr = idx2d.shape[0]
    info = plsc.get_sparse_core_info()
    nw = info.num_cores * info.num_subcores
    rpw = nr // nw
    nch = rpw // CHUNK_IR

    mesh = plsc.VectorSubcoreMesh(core_axis_name="c", subcore_axis_name="s")

    @functools.partial(
        pl.kernel,
        mesh=mesh,
        compiler_params=pltpu.CompilerParams(use_tc_tiling_on_sc=False),
        out_type=jax.ShapeDtypeStruct((nr * IDXW, EMBED), jnp.float32),
        scratch_types=[
            pltpu.VMEM((CHUNK_IR, IDXW), jnp.int32),
            pltpu.VMEM((CHUNK_IR * IDXW, EMBED), jnp.float32),
            pltpu.SemaphoreType.DMA,
        ],
    )
    def k(tab_hbm, idx_hbm, out_hbm, idx_v, rows_v, sem):
        wid = lax.axis_index("s") * info.num_cores + lax.axis_index("c")
        r0 = wid * rpw

        def body(g, carry):
            base = r0 + g * CHUNK_IR
            pltpu.sync_copy(idx_hbm.at[pl.ds(base, CHUNK_IR)], idx_v)
            cps = [
                pltpu.async_copy(
                    tab_hbm.at[idx_v.at[j]],
                    rows_v.at[pl.ds(j * IDXW, IDXW)],
                    sem,
                )
                for j in range(CHUNK_IR)
            ]
            for cp in cps:
                cp.wait()
            pltpu.sync_copy(
                rows_v,
                out_hbm.at[pl.ds(base * IDXW, CHUNK_IR * IDXW)],
            )
            return carry

        lax.fori_loop(0, nch, body, 0)

    return k(table, idx2d)


def _to_row_major(wt):
    vpad = 1 << 20
    band = vpad // 4
    blkv = 2048
    wt = jnp.pad(wt, ((0, 0), (0, vpad - wt.shape[1])))

    def body(x0_ref, x1_ref, x2_ref, x3_ref, o_ref):
        o_ref[...] = jnp.concatenate(
            [jnp.swapaxes(x_ref[...], 0, 1)
             for x_ref in (x0_ref, x1_ref, x2_ref, x3_ref)],
            axis=1,
        )

    nblk = band // blkv

    def in_spec(b):
        return pl.BlockSpec((EMBED, blkv), lambda i, b=b: (0, nblk * b + i))

    return pl.pallas_call(
        body,
        grid=(nblk,),
        in_specs=[in_spec(0), in_spec(1), in_spec(2), in_spec(3)],
        out_specs=pl.BlockSpec((blkv, 128), lambda i: (i, 0)),
        out_shape=jax.ShapeDtypeStruct((band, 128), jnp.float32),
    )(wt, wt, wt, wt)


def _f16_bits(o):
    u = lax.bitcast_convert_type(o, jnp.int32)
    sign = (u >> 16) & 0x8000
    au = u & 0x7FFFFFFF
    b = au - 0x38000000
    r = b >> 13
    low = b & 0x1FFF
    rup = (low > 0x1000) | ((low == 0x1000) & ((r & 1) == 1))
    hn = r + rup.astype(jnp.int32)
    xs = jnp.abs(o) * 16777216.0
    hs = jnp.round(xs).astype(jnp.int32)
    return jnp.where(au < 0x38800000, hs, hn) | sign


def _seg_max32(a):
    n = a.shape[-1]
    lanes = lax.broadcasted_iota(jnp.int32, a.shape, len(a.shape) - 1) % 32
    m = a
    for o in (1, 2, 4, 8, 16):
        r1 = pltpu.roll(m, n - o, axis=len(a.shape) - 1)
        r2 = pltpu.roll(m, 32 - o, axis=len(a.shape) - 1)
        m = jnp.maximum(m, jnp.where(lanes < 32 - o, r1, r2))
    return m


def _dequant(rows4k):
    n = rows4k.shape[0]
    blk = 2048

    def body(x_ref, o_ref):
        w = x_ref[...]
        m = _seg_max32(jnp.abs(w))
        s = jnp.maximum(m, 1e-8) / 127.0
        q = jnp.clip(jnp.round(w / s), -127.0, 127.0)
        t = s * 8193.0
        sf = t - (t - s)
        o = q * sf
        o_ref[...] = _f16_bits(o).astype(jnp.int16)

    return pl.pallas_call(
        body,
        grid=(n // blk,),
        in_specs=[pl.BlockSpec((blk, 128), lambda i: (i, 0))],
        out_specs=pl.BlockSpec((blk, 128), lambda i: (i, 0)),
        out_shape=jax.ShapeDtypeStruct((n, 128), jnp.int16),
    )(rows4k)


def kernel(input_ids, weight):
    bsz, hist = input_ids.shape
    idxp = ((input_ids & 0x3FFFF) << 2) | (input_ids >> 18)
    idx2d = idxp.reshape(-1, IDXW)
    table_lin = _to_row_major(weight.T).reshape(1 << 20, EMBED)
    rows = _gather_rows(table_lin, idx2d)
    bits = _dequant(rows.reshape(-1, 128))
    out = lax.bitcast_convert_type(bits, jnp.float16)
    return out.reshape(bsz, hist, EMBED)
